# baseline (speedup 1.0000x reference)
import functools

import jax
import jax.numpy as jnp
from jax import lax
from jax.experimental import pallas as pl
from jax.experimental.pallas import tpu as pltpu
from jax.experimental.pallas import tpu_sc as plsc

_LANES = 16


def _score_body(x_ref, w2_ref, b2_ref, w1_ref, o_ref):
    x = x_ref[...].astype(jnp.bfloat16)
    proj = lax.dot_general(
        x, w2_ref[...].astype(jnp.bfloat16), (((1,), (1,)), ((), ())),
        preferred_element_type=jnp.float32)
    t = jnp.tanh(proj + b2_ref[...])
    o_ref[...] = jnp.dot(t, w1_ref[...],
                         preferred_element_type=jnp.float32)


def _dense_scores(x, W2, b2, W1, h, n_split):
    M, H = x.shape
    A = W2.shape[0]
    BM = 4096
    blocks = M // BM // n_split
    w1_cols = jnp.broadcast_to(jnp.reshape(W1, (A, 1)), (A, 8))
    out = pl.pallas_call(
        _score_body,
        grid=(blocks,),
        in_specs=[
            pl.BlockSpec((BM, H), lambda m: (m + h * blocks, 0)),
            pl.BlockSpec((A, H), lambda m: (0, 0)),
            pl.BlockSpec((1, A), lambda m: (0, 0)),
            pl.BlockSpec((A, 8), lambda m: (0, 0)),
        ],
        out_specs=pl.BlockSpec((BM, 8), lambda m: (m, 0)),
        out_shape=jax.ShapeDtypeStruct((M // n_split, 8), jnp.float32),
    )(x, W2, jnp.reshape(b2, (1, A)), w1_cols)
    return out[:, 0]


def _fused_sc(lf_flat, score, regions_flat, L, N, R, h):
    M_half = score.shape[0]
    _, H = lf_flat.shape
    G = regions_flat.shape[0] // L
    info = plsc.get_sparse_core_info()
    NC, NS = info.num_cores, info.num_subcores
    NW = NC * NS
    per = G // NW
    n_chunks = H // _LANES

    mesh = plsc.VectorSubcoreMesh(core_axis_name="c", subcore_axis_name="s")

    @functools.partial(
        pl.kernel,
        out_type=jax.ShapeDtypeStruct((G, H), jnp.float32),
        mesh=mesh,
        compiler_params=pltpu.CompilerParams(needs_layout_passes=False),
        scratch_types=[
            pltpu.VMEM((per * L,), jnp.int32),
            pltpu.VMEM((M_half,), jnp.float32),
            pltpu.VMEM((L * per,), jnp.float32),
            pltpu.VMEM((4, L, H), jnp.float32),
            pltpu.VMEM((per, H), jnp.float32),
            pltpu.SemaphoreType.DMA,
            pltpu.SemaphoreType.DMA,
            pltpu.SemaphoreType.DMA,
            pltpu.SemaphoreType.DMA,
        ],
    )
    def k(lf_hbm, score_hbm, idx_hbm, out_hbm,
          idx_v, score_v, w_v, rows_v, out_v,
          sem0, sem1, sem2, sem3):
        wid = lax.axis_index("s") * NC + lax.axis_index("c")
        base = wid * per
        sems = (sem0, sem1, sem2, sem3)

        def fire(r, b):
            pltpu.async_copy(
                lf_hbm.at[idx_v.at[pl.ds(r * L, L)]], rows_v.at[b], sems[b])

        def wait(b):
            pltpu.make_async_copy(
                lf_hbm.at[pl.ds(0, L)], rows_v.at[b], sems[b]).wait()

        pltpu.sync_copy(idx_hbm.at[pl.ds(base * L, per * L)], idx_v)

        bN = h * M_half + (base // R) * N

        @plsc.parallel_loop(0, per * L // _LANES, 1, unroll=4)
        def idx_body(c):
            sl = pl.ds(c * _LANES, _LANES)
            idx_v[sl] = idx_v[sl] + bN

        fire(0, 0)
        fire(1, 1)
        fire(2, 2)
        pltpu.sync_copy(score_hbm, score_v)

        for g in range(per // _LANES):
            r_vec = g * _LANES + lax.iota(jnp.int32, _LANES)
            a = []
            for l in range(L):
                ii = plsc.load_gather(idx_v, [r_vec * L + l])
                a.append(plsc.load_gather(score_v, [ii - (h * M_half)]))
            m = a[0]
            for l in range(1, L):
                m = jnp.maximum(m, a[l])
            e = [jnp.exp(av - m) for av in a]
            s = e[0]
            for l in range(1, L):
                s = s + e[l]
            inv = 1.0 / s
            for l in range(L):
                w_v[pl.ds(l * per + g * _LANES, _LANES)] = e[l] * inv

        def compute(r, b):
            w = [
                plsc.load_gather(
                    w_v, [jnp.full((_LANES,), l * per, jnp.int32) + r])
                for l in range(L)
            ]

            @plsc.parallel_loop(0, n_chunks, 1, unroll=8)
            def chunk_body(c):
                sl = pl.ds(c * _LANES, _LANES)
                t0 = w[0] * rows_v[b, 0, sl] + w[1] * rows_v[b, 1, sl]
                t1 = w[2] * rows_v[b, 2, sl] + w[3] * rows_v[b, 3, sl]
                t2 = w[4] * rows_v[b, 4, sl] + w[5] * rows_v[b, 5, sl]
                t3 = w[6] * rows_v[b, 6, sl] + w[7] * rows_v[b, 7, sl]
                out_v[r, sl] = (t0 + t1) + (t2 + t3)

        def body(i, carry):
            r0 = 4 * i
            for b in range(4):
                r = r0 + b
                wait(b)
                compute(r, b)
                fire(r + 3, (b + 3) % 4)
            return carry

        lax.fori_loop(0, per // 4 - 1, body, 0)
        r0 = per - 4
        wait(0)
        compute(r0, 0)
        fire(per - 1, 3)
        for b in range(1, 4):
            wait(b)
            compute(r0 + b, b)

        pltpu.sync_copy(out_v, out_hbm.at[pl.ds(base, per)])

    return k(lf_flat, score, regions_flat)


def kernel(line_features, regions, W2, b2, W1, b1):
    B, N, H = line_features.shape
    _, R, L = regions.shape
    x = line_features.reshape(B * N, H)
    regions_flat = regions.reshape(B * R * L)
    n_split = 2
    gl = B * R * L // n_split
    halves = []
    for h in range(n_split):
        score_h = _dense_scores(x, W2, b2, W1, h, n_split)
        halves.append(_fused_sc(
            x, score_h, regions_flat[h * gl:(h + 1) * gl], L, N, R, h))
    fused = jnp.concatenate(halves, axis=0)
    fused = fused.reshape(B, R, H)
    region_mask = jnp.ones((B, R), dtype=bool)
    return (fused, region_mask)


# speedup vs baseline: 1.0319x; 1.0319x over previous
import functools

import jax
import jax.numpy as jnp
from jax import lax
from jax.experimental import pallas as pl
from jax.experimental.pallas import tpu as pltpu
from jax.experimental.pallas import tpu_sc as plsc

_LANES = 16


def _score_body(x_ref, w2_ref, b2_ref, w1_ref, o_ref):
    x = x_ref[...].astype(jnp.bfloat16)
    proj = lax.dot_general(
        x, w2_ref[...].astype(jnp.bfloat16), (((1,), (1,)), ((), ())),
        preferred_element_type=jnp.float32)
    t = jnp.tanh(proj + b2_ref[...])
    o_ref[...] = jnp.dot(t, w1_ref[...],
                         preferred_element_type=jnp.float32)


def _dense_scores(x, W2, b2, W1, h, n_split):
    M, H = x.shape
    A = W2.shape[0]
    BM = 2048
    blocks = M // BM // n_split
    w1_cols = jnp.broadcast_to(jnp.reshape(W1, (A, 1)), (A, 8))
    out = pl.pallas_call(
        _score_body,
        grid=(blocks,),
        in_specs=[
            pl.BlockSpec((BM, H), lambda m: (m + h * blocks, 0)),
            pl.BlockSpec((A, H), lambda m: (0, 0)),
            pl.BlockSpec((1, A), lambda m: (0, 0)),
            pl.BlockSpec((A, 8), lambda m: (0, 0)),
        ],
        out_specs=pl.BlockSpec((BM, 8), lambda m: (m, 0)),
        out_shape=jax.ShapeDtypeStruct((M // n_split, 8), jnp.float32),
    )(x, W2, jnp.reshape(b2, (1, A)), w1_cols)
    return out[:, 0]


def _fused_sc(lf_flat, score, regions_flat, L, N, R, h):
    M_half = score.shape[0]
    _, H = lf_flat.shape
    G = regions_flat.shape[0] // L
    info = plsc.get_sparse_core_info()
    NC, NS = info.num_cores, info.num_subcores
    NW = NC * NS
    per = G // NW
    n_chunks = H // _LANES

    mesh = plsc.VectorSubcoreMesh(core_axis_name="c", subcore_axis_name="s")

    @functools.partial(
        pl.kernel,
        out_type=jax.ShapeDtypeStruct((G, H), jnp.float32),
        mesh=mesh,
        compiler_params=pltpu.CompilerParams(needs_layout_passes=False),
        scratch_types=[
            pltpu.VMEM((per * L,), jnp.int32),
            pltpu.VMEM((M_half,), jnp.float32),
            pltpu.VMEM((L * per,), jnp.float32),
            pltpu.VMEM((4, L, H), jnp.float32),
            pltpu.VMEM((per, H), jnp.float32),
            pltpu.SemaphoreType.DMA,
            pltpu.SemaphoreType.DMA,
            pltpu.SemaphoreType.DMA,
            pltpu.SemaphoreType.DMA,
        ],
    )
    def k(lf_hbm, score_hbm, idx_hbm, out_hbm,
          idx_v, score_v, w_v, rows_v, out_v,
          sem0, sem1, sem2, sem3):
        wid = lax.axis_index("s") * NC + lax.axis_index("c")
        base = wid * per
        sems = (sem0, sem1, sem2, sem3)

        def fire(r, b):
            pltpu.async_copy(
                lf_hbm.at[idx_v.at[pl.ds(r * L, L)]], rows_v.at[b], sems[b])

        def wait(b):
            pltpu.make_async_copy(
                lf_hbm.at[pl.ds(0, L)], rows_v.at[b], sems[b]).wait()

        pltpu.sync_copy(idx_hbm.at[pl.ds(base * L, per * L)], idx_v)

        bN = h * M_half + (base // R) * N

        @plsc.parallel_loop(0, per * L // _LANES, 1, unroll=4)
        def idx_body(c):
            sl = pl.ds(c * _LANES, _LANES)
            idx_v[sl] = idx_v[sl] + bN

        fire(0, 0)
        fire(1, 1)
        fire(2, 2)
        pltpu.sync_copy(score_hbm, score_v)

        for g in range(per // _LANES):
            r_vec = g * _LANES + lax.iota(jnp.int32, _LANES)
            a = []
            for l in range(L):
                ii = plsc.load_gather(idx_v, [r_vec * L + l])
                a.append(plsc.load_gather(score_v, [ii - (h * M_half)]))
            m = a[0]
            for l in range(1, L):
                m = jnp.maximum(m, a[l])
            e = [jnp.exp(av - m) for av in a]
            s = e[0]
            for l in range(1, L):
                s = s + e[l]
            inv = 1.0 / s
            for l in range(L):
                w_v[pl.ds(l * per + g * _LANES, _LANES)] = e[l] * inv

        def compute(r, b):
            w = [
                plsc.load_gather(
                    w_v, [jnp.full((_LANES,), l * per, jnp.int32) + r])
                for l in range(L)
            ]

            @plsc.parallel_loop(0, n_chunks, 1, unroll=8)
            def chunk_body(c):
                sl = pl.ds(c * _LANES, _LANES)
                t0 = w[0] * rows_v[b, 0, sl] + w[1] * rows_v[b, 1, sl]
                t1 = w[2] * rows_v[b, 2, sl] + w[3] * rows_v[b, 3, sl]
                t2 = w[4] * rows_v[b, 4, sl] + w[5] * rows_v[b, 5, sl]
                t3 = w[6] * rows_v[b, 6, sl] + w[7] * rows_v[b, 7, sl]
                out_v[r, sl] = (t0 + t1) + (t2 + t3)

        def body(i, carry):
            r0 = 4 * i
            for b in range(4):
                r = r0 + b
                wait(b)
                compute(r, b)
                fire(r + 3, (b + 3) % 4)
            return carry

        lax.fori_loop(0, per // 4 - 1, body, 0)
        r0 = per - 4
        wait(0)
        compute(r0, 0)
        fire(per - 1, 3)
        for b in range(1, 4):
            wait(b)
            compute(r0 + b, b)

        pltpu.sync_copy(out_v, out_hbm.at[pl.ds(base, per)])

    return k(lf_flat, score, regions_flat)


def kernel(line_features, regions, W2, b2, W1, b1):
    B, N, H = line_features.shape
    _, R, L = regions.shape
    x = line_features.reshape(B * N, H)
    regions_flat = regions.reshape(B * R * L)
    n_split = 2
    gl = B * R * L // n_split
    halves = []
    for h in range(n_split):
        score_h = _dense_scores(x, W2, b2, W1, h, n_split)
        halves.append(_fused_sc(
            x, score_h, regions_flat[h * gl:(h + 1) * gl], L, N, R, h))
    fused = jnp.concatenate(halves, axis=0)
    fused = fused.reshape(B, R, H)
    region_mask = jnp.ones((B, R), dtype=bool)
    return (fused, region_mask)


# speedup vs baseline: 1.0465x; 1.0141x over previous
import functools

import jax
import jax.numpy as jnp
from jax import lax
from jax.experimental import pallas as pl
from jax.experimental.pallas import tpu as pltpu
from jax.experimental.pallas import tpu_sc as plsc

_LANES = 16


def _score_body(x_ref, w2_ref, b2_ref, w1_ref, o_ref):
    x = x_ref[...].astype(jnp.bfloat16)
    proj = lax.dot_general(
        x, w2_ref[...].astype(jnp.bfloat16), (((1,), (1,)), ((), ())),
        preferred_element_type=jnp.float32)
    t = jnp.tanh(proj + b2_ref[...])
    o_ref[...] = jnp.dot(t, w1_ref[...],
                         preferred_element_type=jnp.float32)


def _dense_scores(x, W2, b2, W1, h, n_split):
    M, H = x.shape
    A = W2.shape[0]
    BM = 2048
    blocks = M // BM // n_split
    w1_cols = jnp.broadcast_to(jnp.reshape(W1, (A, 1)), (A, 8))
    out = pl.pallas_call(
        _score_body,
        grid=(blocks,),
        in_specs=[
            pl.BlockSpec((BM, H), lambda m: (m + h * blocks, 0)),
            pl.BlockSpec((A, H), lambda m: (0, 0)),
            pl.BlockSpec((1, A), lambda m: (0, 0)),
            pl.BlockSpec((A, 8), lambda m: (0, 0)),
        ],
        out_specs=pl.BlockSpec((BM, 8), lambda m: (m, 0)),
        out_shape=jax.ShapeDtypeStruct((M // n_split, 8), jnp.float32),
    )(x, W2, jnp.reshape(b2, (1, A)), w1_cols)
    return out[:, 0]


def _fused_sc(lf_flat, score, regions_flat, L, N, R, h):
    M_half = score.shape[0]
    _, H = lf_flat.shape
    G = regions_flat.shape[0] // L
    info = plsc.get_sparse_core_info()
    NC, NS = info.num_cores, info.num_subcores
    NW = NC * NS
    per = G // NW
    n_chunks = H // _LANES

    mesh = plsc.VectorSubcoreMesh(core_axis_name="c", subcore_axis_name="s")

    @functools.partial(
        pl.kernel,
        out_type=jax.ShapeDtypeStruct((G, H), jnp.float32),
        mesh=mesh,
        compiler_params=pltpu.CompilerParams(needs_layout_passes=False),
        scratch_types=[
            pltpu.VMEM((per * L,), jnp.int32),
            pltpu.VMEM((M_half,), jnp.float32),
            pltpu.VMEM((L * per,), jnp.float32),
            pltpu.VMEM((4, L, H), jnp.float32),
            pltpu.VMEM((per, H), jnp.float32),
            pltpu.SemaphoreType.DMA,
            pltpu.SemaphoreType.DMA,
            pltpu.SemaphoreType.DMA,
            pltpu.SemaphoreType.DMA,
        ],
    )
    def k(lf_hbm, score_hbm, idx_hbm, out_hbm,
          idx_v, score_v, w_v, rows_v, out_v,
          sem0, sem1, sem2, sem3):
        wid = lax.axis_index("s") * NC + lax.axis_index("c")
        base = wid * per
        sems = (sem0, sem1, sem2, sem3)

        def fire(r, b):
            pltpu.async_copy(
                lf_hbm.at[idx_v.at[pl.ds(r * L, L)]], rows_v.at[b], sems[b])

        def wait(b):
            pltpu.make_async_copy(
                lf_hbm.at[pl.ds(0, L)], rows_v.at[b], sems[b]).wait()

        pltpu.sync_copy(idx_hbm.at[pl.ds(base * L, per * L)], idx_v)

        bN = h * M_half + (base // R) * N

        @plsc.parallel_loop(0, per * L // _LANES, 1, unroll=4)
        def idx_body(c):
            sl = pl.ds(c * _LANES, _LANES)
            idx_v[sl] = idx_v[sl] + bN

        fire(0, 0)
        fire(1, 1)
        fire(2, 2)
        pltpu.sync_copy(score_hbm, score_v)

        for g in range(per // _LANES):
            r_vec = g * _LANES + lax.iota(jnp.int32, _LANES)
            a = []
            for l in range(L):
                ii = plsc.load_gather(idx_v, [r_vec * L + l])
                a.append(plsc.load_gather(score_v, [ii - (h * M_half)]))
            m = a[0]
            for l in range(1, L):
                m = jnp.maximum(m, a[l])
            e = [jnp.exp(av - m) for av in a]
            s = e[0]
            for l in range(1, L):
                s = s + e[l]
            inv = 1.0 / s
            for l in range(L):
                w_v[pl.ds(l * per + g * _LANES, _LANES)] = e[l] * inv

        def compute(r, b):
            w = [
                plsc.load_gather(
                    w_v, [jnp.full((_LANES,), l * per, jnp.int32) + r])
                for l in range(L)
            ]

            @plsc.parallel_loop(0, n_chunks, 1, unroll=8)
            def chunk_body(c):
                sl = pl.ds(c * _LANES, _LANES)
                t0 = w[0] * rows_v[b, 0, sl] + w[1] * rows_v[b, 1, sl]
                t1 = w[2] * rows_v[b, 2, sl] + w[3] * rows_v[b, 3, sl]
                t2 = w[4] * rows_v[b, 4, sl] + w[5] * rows_v[b, 5, sl]
                t3 = w[6] * rows_v[b, 6, sl] + w[7] * rows_v[b, 7, sl]
                out_v[r, sl] = (t0 + t1) + (t2 + t3)

        def body(i, carry):
            r0 = 4 * i
            for b in range(4):
                r = r0 + b
                wait(b)
                fire(r + 3, (b + 3) % 4)
                compute(r, b)
            return carry

        lax.fori_loop(0, per // 4 - 1, body, 0)
        r0 = per - 4
        wait(0)
        fire(per - 1, 3)
        compute(r0, 0)
        for b in range(1, 4):
            wait(b)
            compute(r0 + b, b)

        pltpu.sync_copy(out_v, out_hbm.at[pl.ds(base, per)])

    return k(lf_flat, score, regions_flat)


def kernel(line_features, regions, W2, b2, W1, b1):
    B, N, H = line_features.shape
    _, R, L = regions.shape
    x = line_features.reshape(B * N, H)
    regions_flat = regions.reshape(B * R * L)
    n_split = 2
    gl = B * R * L // n_split
    halves = []
    for h in range(n_split):
        score_h = _dense_scores(x, W2, b2, W1, h, n_split)
        halves.append(_fused_sc(
            x, score_h, regions_flat[h * gl:(h + 1) * gl], L, N, R, h))
    fused = jnp.concatenate(halves, axis=0)
    fused = fused.reshape(B, R, H)
    region_mask = jnp.ones((B, R), dtype=bool)
    return (fused, region_mask)
